# baseline (speedup 1.0000x reference)
import functools

import jax
import jax.numpy as jnp
from jax import lax
from jax.experimental import pallas as pl
from jax.experimental.pallas import tpu as pltpu
from jax.experimental.pallas import tpu_sc as plsc

E = 64
B = 4096
S = 200
NC = 2
NSUB = 16
NW = NC * NSUB
BBLK = B // NW
NS = 2
NCHUNK = S // NS
L = 16
RP = E + 1


@jax.jit
def _run(xt, table, pos):
    mesh = plsc.VectorSubcoreMesh(core_axis_name="c", subcore_axis_name="s")

    @functools.partial(
        pl.kernel,
        mesh=mesh,
        compiler_params=pltpu.CompilerParams(
            use_tc_tiling_on_sc=False, needs_layout_passes=False,
            disable_bounds_checks=True),
        out_type=jax.ShapeDtypeStruct((S, E // 8, NW, 8, BBLK), jnp.float32),
        scratch_types=[
            pltpu.VMEM((2, NS, BBLK), jnp.int32),
            pltpu.VMEM((NS * BBLK, E), jnp.float32),
            pltpu.VMEM((NS * BBLK, E), jnp.float32),
            pltpu.VMEM((NS * BBLK, RP), jnp.float32),
            pltpu.VMEM((NS, E // 8, 8, BBLK), jnp.float32),
            pltpu.VMEM((S, E), jnp.float32),
            pltpu.SemaphoreType.DMA,
            pltpu.SemaphoreType.DMA,
            pltpu.SemaphoreType.DMA,
            pltpu.SemaphoreType.DMA,
        ],
    )
    def body(xt_hbm, table_hbm, pos_hbm, out_hbm,
             idx_v, rows0_v, rows1_v, rp_v, stage_v, pos_v,
             sem0, sem1, sem_idx, sem_out):
        wid = lax.axis_index("s") * NC + lax.axis_index("c")
        rows_bufs = (rows0_v, rows1_v)
        sems = (sem0, sem1)
        pltpu.sync_copy(pos_hbm, pos_v)

        def idx_src(c):
            return xt_hbm.at[pl.ds(c * NS, NS), pl.ds(wid * BBLK, BBLK)]

        def fire(buf):
            for j in range(NS):
                pltpu.async_copy(
                    table_hbm.at[idx_v.at[buf, j]],
                    rows_bufs[buf].at[pl.ds(j * BBLK, BBLK)],
                    sems[buf])

        def drain(buf):
            for j in range(NS):
                pltpu.make_async_copy(
                    table_hbm.at[idx_v.at[buf, j]],
                    rows_bufs[buf].at[pl.ds(j * BBLK, BBLK)],
                    sems[buf]).wait()

        def out_dst(c):
            return out_hbm.at[pl.ds(c * NS, NS), :, wid]

        pltpu.sync_copy(idx_src(0), idx_v.at[0])
        fire(0)
        pltpu.async_copy(idx_src(1), idx_v.at[1], sem_idx)

        lanes = lax.iota(jnp.int32, L)
        zv = lanes * 0

        def chunk_pair(c, carry):
            for b in range(2):
                cc = c + b
                nb = 1 - b

                @pl.when(cc + 1 < NCHUNK)
                def _():
                    pltpu.make_async_copy(
                        idx_src(cc + 1), idx_v.at[nb], sem_idx).wait()
                    fire(nb)

                drain(b)

                @pl.when(cc + 2 < NCHUNK)
                def _():
                    pltpu.async_copy(idx_src(cc + 2), idx_v.at[b], sem_idx)

                @pl.when(cc > 0)
                def _():
                    pltpu.make_async_copy(
                        stage_v, out_dst(cc - 1), sem_out).wait()

                rows = rows_bufs[b]

                for j in range(NS):
                    pv = [pos_v[cc * NS + j, pl.ds(k * L, L)]
                          for k in range(E // L)]

                    def r_body(i, carry2, j=j, pv=pv):
                        for u in range(4):
                            r = j * BBLK + i * 4 + u
                            for k in range(E // L):
                                rp_v[r, pl.ds(k * L, L)] = (
                                    rows[r, pl.ds(k * L, L)] + pv[k])
                        return carry2

                    lax.fori_loop(0, BBLK // 4, r_body, 0)

                def t_body(i, carry2):
                    j = i // (BBLK // L)
                    q = i % (BBLK // L)
                    rowv = lanes + (j * BBLK + q * L)
                    for e in range(E):
                        col = plsc.load_gather(rp_v, [rowv, zv + e])
                        stage_v[j, e // 8, e % 8, pl.ds(q * L, L)] = col
                    return carry2

                lax.fori_loop(0, NS * (BBLK // L), t_body, 0)

                pltpu.async_copy(stage_v, out_dst(cc), sem_out)
            return carry

        lax.fori_loop(0, NCHUNK // 2, lambda i, c: chunk_pair(i * 2, c), 0)
        pltpu.make_async_copy(
            stage_v, out_dst(NCHUNK - 1), sem_out).wait()

    return body(xt, table, pos)


def kernel(x, table, pos_encoding):
    xt = x.T
    pos = pos_encoding[:S]
    a = _run(xt, table, pos)
    return a.transpose(2, 4, 0, 1, 3).reshape(B, S, E)


# speedup vs baseline: 1.4809x; 1.4809x over previous
import functools

import jax
import jax.numpy as jnp
from jax import lax
from jax.experimental import pallas as pl
from jax.experimental.pallas import tpu as pltpu
from jax.experimental.pallas import tpu_sc as plsc

E = 64
B = 4096
S = 200
NC = 2
NSUB = 16
NW = NC * NSUB
BBLK = B // NW
NS = 4
NCHUNK = S // NS
L = 16
SP = BBLK + 1


@jax.jit
def _run(xt, table, pos):
    mesh = plsc.VectorSubcoreMesh(core_axis_name="c", subcore_axis_name="s")

    @functools.partial(
        pl.kernel,
        mesh=mesh,
        compiler_params=pltpu.CompilerParams(
            use_tc_tiling_on_sc=False, needs_layout_passes=False,
            disable_bounds_checks=True),
        out_type=jax.ShapeDtypeStruct((S, E // 8, NW, 8, BBLK), jnp.float32),
        scratch_types=[
            pltpu.VMEM((2, NS * BBLK), jnp.int32),
            pltpu.VMEM((NS * BBLK, E), jnp.float32),
            pltpu.VMEM((NS * BBLK, E), jnp.float32),
            pltpu.VMEM((NS, E // 8, 8, SP), jnp.float32),
            pltpu.VMEM((S, E), jnp.float32),
            pltpu.SemaphoreType.DMA,
            pltpu.SemaphoreType.DMA,
            pltpu.SemaphoreType.DMA,
            pltpu.SemaphoreType.DMA,
        ],
    )
    def body(xt_hbm, table_hbm, pos_hbm, out_hbm,
             idx_v, rows0_v, rows1_v, stage_v, pos_v,
             sem0, sem1, sem_idx, sem_out):
        wid = lax.axis_index("s") * NC + lax.axis_index("c")
        rows_bufs = (rows0_v, rows1_v)
        sems = (sem0, sem1)
        pltpu.sync_copy(pos_hbm, pos_v)

        def idx_fetch(c, buf, sem):
            for j in range(NS):
                pltpu.async_copy(
                    xt_hbm.at[c * NS + j, pl.ds(wid * BBLK, BBLK)],
                    idx_v.at[buf, pl.ds(j * BBLK, BBLK)], sem)

        def idx_wait(c, buf, sem):
            for j in range(NS):
                pltpu.make_async_copy(
                    xt_hbm.at[c * NS + j, pl.ds(wid * BBLK, BBLK)],
                    idx_v.at[buf, pl.ds(j * BBLK, BBLK)], sem).wait()

        def fire(c, buf):
            pltpu.async_copy(
                table_hbm.at[idx_v.at[buf]], rows_bufs[buf], sems[buf])

        def drain(buf):
            pltpu.make_async_copy(
                table_hbm.at[idx_v.at[buf]], rows_bufs[buf],
                sems[buf]).wait()

        def out_dst(c):
            return out_hbm.at[pl.ds(c * NS, NS), :, wid]

        idx_fetch(0, 0, sem_idx)
        idx_wait(0, 0, sem_idx)
        fire(0, 0)
        idx_fetch(1, 1, sem_idx)

        lanes = lax.iota(jnp.int32, L)
        zv = lanes * 0

        def chunk_pair(c, carry):
            for b in range(2):
                cc = c + b
                nb = 1 - b

                @pl.when(cc + 1 < NCHUNK)
                def _():
                    idx_wait(cc + 1, nb, sem_idx)
                    fire(cc + 1, nb)

                drain(b)

                @pl.when(cc + 2 < NCHUNK)
                def _():
                    idx_fetch(cc + 2, b, sem_idx)

                @pl.when(cc > 0)
                def _():
                    pltpu.make_async_copy(
                        stage_v.at[:, :, :, pl.ds(0, BBLK)],
                        out_dst(cc - 1), sem_out).wait()

                rows = rows_bufs[b]

                def jb_body(i, carry2):
                    j = i // (BBLK // 8)
                    i8 = i % (BBLK // 8)
                    jev = [j * E + k * L + lanes for k in range(E // L)]
                    pv = [pos_v[cc * NS + j, pl.ds(k * L, L)]
                          for k in range(E // L)]
                    for bu in range(8):
                        bb = i8 * 8 + bu
                        bv = jnp.full((L,), bb, jnp.int32)
                        for k in range(E // L):
                            vec = (rows[j * BBLK + bb, pl.ds(k * L, L)]
                                   + pv[k])
                            plsc.store_scatter(
                                stage_v, [zv, zv, jev[k], bv], vec)
                    return carry2

                lax.fori_loop(0, NS * (BBLK // 8), jb_body, 0)

                pltpu.async_copy(
                    stage_v.at[:, :, :, pl.ds(0, BBLK)],
                    out_dst(cc), sem_out)
            return carry

        lax.fori_loop(0, NCHUNK // 2, lambda i, c: chunk_pair(i * 2, c), 0)
        pltpu.make_async_copy(
            stage_v.at[:, :, :, pl.ds(0, BBLK)],
            out_dst(NCHUNK - 1), sem_out).wait()

    return body(xt, table, pos)


def kernel(x, table, pos_encoding):
    xt = x.T
    pos = pos_encoding[:S]
    a = _run(xt, table, pos)
    return a.transpose(2, 4, 0, 1, 3).reshape(B, S, E)
